# Initial kernel scaffold; baseline (speedup 1.0000x reference)
import jax
import jax.numpy as jnp
from jax.experimental import pallas as pl


def kernel(x, W1, W2, W3, W4, bn_weight, bn_bias):
    raise NotImplementedError("write your pallas kernel here")


# speedup vs baseline: 1.0765x; 1.0765x over previous
import functools

import jax
import jax.numpy as jnp
from jax.experimental import pallas as pl
from jax.experimental.pallas import tpu as pltpu

_GRID = 10
_NSEG = _GRID * _GRID
_INTERVAL = 0.2
_EPS = 1e-5
_N = 16384
_CH = 2048
_NCHUNK = _N // _CH
_NEG = -1e30


def _encode_body(x_ref, w1_ref, w2_ref, w3_ref, w4a_ref, w4b_ref,
                 enc_ref, acc_ref, cnt_ref, cen_ref):
    f32 = jnp.float32

    def coords(c):
        xc = x_ref[0, pl.ds(c * _CH, _CH), :]
        y = xc[:, 0:1]
        z = xc[:, 1:2]
        xx = xc[:, 2:3]
        swapped = jnp.concatenate([z, y, xx], axis=1)
        ybin = jnp.floor(jnp.clip(y + 1.0, 0.0, 1.99) / _INTERVAL)
        xbin = jnp.floor(jnp.clip(xx + 1.0, 0.0, 1.99) / _INTERVAL)
        pid = (ybin * float(_GRID) + xbin).astype(jnp.int32)
        iota = jax.lax.broadcasted_iota(jnp.int32, (_CH, _NSEG), 1)
        onehot = (iota == pid).astype(f32)
        return swapped, pid, onehot

    cnt = jnp.zeros((_NSEG, 1), f32)
    sums = jnp.zeros((_NSEG, 3), f32)
    for c in range(_NCHUNK):
        swapped, _, onehot = coords(c)
        ones = jnp.ones((_CH, 1), f32)
        cnt = cnt + jax.lax.dot_general(
            onehot, ones, (((0,), (0,)), ((), ())),
            preferred_element_type=f32)
        sums = sums + jax.lax.dot_general(
            onehot, swapped, (((0,), (0,)), ((), ())),
            preferred_element_type=f32)
    cnt_ref[...] = cnt
    cen_ref[...] = sums / jnp.maximum(cnt, 1.0)

    acc_ref[...] = jnp.full((_NSEG, 128 + 768), _NEG, f32)
    for c in range(_NCHUNK):
        swapped, pid, onehot = coords(c)
        cen_pts = jnp.dot(onehot, cen_ref[...], preferred_element_type=f32)
        aug = jnp.concatenate([swapped, swapped - cen_pts], axis=1)
        h = jax.nn.relu(jnp.dot(aug, w1_ref[...], preferred_element_type=f32))
        h = jax.nn.relu(jnp.dot(h, w2_ref[...], preferred_element_type=f32))
        h = jax.nn.relu(jnp.dot(h, w3_ref[...], preferred_element_type=f32))
        g = jnp.dot(h, w4a_ref[...], preferred_element_type=f32)
        hg = jnp.concatenate([h, g], axis=1)

        def seg_step(s, _):
            m = pid == s
            cand = jnp.max(jnp.where(m, hg, _NEG), axis=0, keepdims=True)
            acc_ref[pl.ds(s, 1), :] = jnp.maximum(acc_ref[pl.ds(s, 1), :], cand)
            return 0

        jax.lax.fori_loop(0, _NSEG, seg_step, 0)

    overall = jnp.maximum(acc_ref[:, :128], 0.0)
    m_part = acc_ref[:, 128:]
    cvec = jnp.dot(overall, w4b_ref[...], preferred_element_type=f32)
    enc = jax.nn.relu(m_part + cvec)
    occ = cnt_ref[...] > 0.0
    enc_ref[0] = jnp.where(occ, enc, 0.0)


def _bn_body(enc_ref, w_ref, b_ref, out_ref):
    f32 = jnp.float32
    bsz = enc_ref.shape[0]
    nelem = float(bsz * enc_ref.shape[2])
    s1 = jnp.zeros((_NSEG, 1), f32)
    s2 = jnp.zeros((_NSEG, 1), f32)
    for b in range(bsz):
        e = enc_ref[b]
        s1 = s1 + jnp.sum(e, axis=1, keepdims=True)
        s2 = s2 + jnp.sum(e * e, axis=1, keepdims=True)
    mu = s1 / nelem
    var = s2 / nelem - mu * mu
    scale = w_ref[...] * jax.lax.rsqrt(var + _EPS)
    shift = b_ref[...] - mu * scale
    for b in range(bsz):
        out_ref[b] = enc_ref[b] * scale + shift


@jax.jit
def kernel(x, W1, W2, W3, W4, bn_weight, bn_bias):
    bsz = x.shape[0]
    w4a = W4[:128]
    w4b = W4[128:]
    enc = pl.pallas_call(
        _encode_body,
        grid=(bsz,),
        in_specs=[
            pl.BlockSpec((1, _N, 3), lambda b: (b, 0, 0)),
            pl.BlockSpec((6, 32), lambda b: (0, 0)),
            pl.BlockSpec((32, 64), lambda b: (0, 0)),
            pl.BlockSpec((64, 128), lambda b: (0, 0)),
            pl.BlockSpec((128, 768), lambda b: (0, 0)),
            pl.BlockSpec((128, 768), lambda b: (0, 0)),
        ],
        out_specs=pl.BlockSpec((1, _NSEG, 768), lambda b: (b, 0, 0)),
        out_shape=jax.ShapeDtypeStruct((bsz, _NSEG, 768), jnp.float32),
        scratch_shapes=[
            pltpu.VMEM((_NSEG, 128 + 768), jnp.float32),
            pltpu.VMEM((_NSEG, 1), jnp.float32),
            pltpu.VMEM((_NSEG, 3), jnp.float32),
        ],
    )(x, W1, W2, W3, w4a, w4b)

    out = pl.pallas_call(
        _bn_body,
        out_shape=jax.ShapeDtypeStruct((bsz, _NSEG, 768), jnp.float32),
    )(enc, bn_weight.reshape(_NSEG, 1), bn_bias.reshape(_NSEG, 1))
    return out


# speedup vs baseline: 3.2131x; 2.9848x over previous
import functools

import jax
import jax.numpy as jnp
from jax.experimental import pallas as pl
from jax.experimental.pallas import tpu as pltpu
from jax.experimental.pallas import tpu_sc as plsc

_GRID = 10
_NSEG = _GRID * _GRID
_INTERVAL = 0.2
_EPS = 1e-5
_B = 8
_N = 16384
_TOT = _B * _N
_NEG = -1e30

_PCH = 512
_NPCH = _N // _PCH
_CH = 2048
_NCHUNK = _N // _CH

_NW = 32
_RPW = _TOT // _NW
_IDXROWS_PW = _RPW // 128
_PW = 16


def _bins(xc):
    f32 = jnp.float32
    y = xc[:, 0:1]
    z = xc[:, 1:2]
    xx = xc[:, 2:3]
    swapped = jnp.concatenate([z, y, xx], axis=1)
    ybin = jnp.floor(jnp.clip(y + 1.0, 0.0, 1.99) / _INTERVAL)
    xbin = jnp.floor(jnp.clip(xx + 1.0, 0.0, 1.99) / _INTERVAL)
    pidf = ybin * float(_GRID) + xbin
    pidi = pidf.astype(jnp.int32)
    n = xc.shape[0]
    iota = jax.lax.broadcasted_iota(jnp.int32, (n, _NSEG), 1)
    onehot = (iota == pidi).astype(f32)
    return swapped, pidf, pidi, onehot


_HI = jax.lax.Precision.HIGHEST


def _chunk_cnt(onehot):
    f32 = jnp.float32
    ones = jnp.ones((_PCH, 1), f32)
    return jnp.round(jax.lax.dot_general(
        onehot, ones, (((0,), (0,)), ((), ())),
        precision=_HI, preferred_element_type=f32))


def _prep1_body(x_ref, packed_ref, cen_ref, cnt_ref, seg_ref, bnd_ref,
                base_ref, sums_ref):
    f32 = jnp.float32
    c = pl.program_id(1)

    swapped, pidf, _, onehot = _bins(x_ref[0])
    packed_ref[0] = jnp.concatenate(
        [swapped, pidf, jnp.zeros((_PCH, _PW - 4), f32)], axis=1)
    chunk_cnt = _chunk_cnt(onehot)
    chunk_sums = jax.lax.dot_general(
        onehot, swapped, (((0,), (0,)), ((), ())), preferred_element_type=f32)

    @pl.when(c == 0)
    def _():
        base_ref[...] = jnp.zeros((_NSEG, 1), f32)
        sums_ref[...] = jnp.zeros((_NSEG, 3), f32)

    base_ref[...] += chunk_cnt
    sums_ref[...] += chunk_sums

    @pl.when(c == _NPCH - 1)
    def _():
        base = base_ref[...]
        cnt_ref[0] = base
        cen_ref[0] = sums_ref[...] / jnp.maximum(base, 1.0)
        ri100 = jax.lax.broadcasted_iota(jnp.int32, (_NSEG, _NSEG), 0)
        ci100 = jax.lax.broadcasted_iota(jnp.int32, (_NSEG, _NSEG), 1)
        lt100 = (ci100 < ri100).astype(f32)
        seg_start = jnp.round(jnp.dot(lt100, base, precision=_HI,
                                      preferred_element_type=f32))
        seg_ref[0] = seg_start
        ends_i = (seg_start + base).astype(jnp.int32)
        pvec = jax.lax.broadcasted_iota(jnp.int32, (1, _NCHUNK), 1) * _CH
        los = jnp.sum((ends_i <= pvec).astype(jnp.int32), axis=0,
                      keepdims=True)
        his = jnp.sum((ends_i <= pvec + (_CH - 1)).astype(jnp.int32), axis=0,
                      keepdims=True)
        bnd_ref[0, 0:1, :] = los
        bnd_ref[0, 1:2, :] = his


def _prep2_body(x_ref, seg_ref, dest_ref, base_ref):
    f32 = jnp.float32
    b = pl.program_id(0)
    c = pl.program_id(1)

    _, _, _, onehot = _bins(x_ref[0])

    @pl.when(c == 0)
    def _():
        base_ref[...] = jnp.zeros((_NSEG, 1), f32)

    ri = jax.lax.broadcasted_iota(jnp.int32, (_PCH, _PCH), 0)
    ci = jax.lax.broadcasted_iota(jnp.int32, (_PCH, _PCH), 1)
    lt = (ci < ri).astype(f32)
    prefix = jnp.dot(lt, onehot, precision=_HI, preferred_element_type=f32)
    inrank = jnp.round(jnp.sum(prefix * onehot, axis=1, keepdims=True))
    baseg = jnp.round(jnp.dot(onehot, base_ref[...] + seg_ref[0],
                              precision=_HI, preferred_element_type=f32))
    dest_ref[0] = (inrank + baseg).astype(jnp.int32) + b * _N
    base_ref[...] += _chunk_cnt(onehot)


def _sc_permute_body(src_hbm, idx_hbm, out_hbm, rows_v, idx_v):
    wid = jax.lax.axis_index("s") * 2 + jax.lax.axis_index("c")
    base = wid * _RPW
    pltpu.sync_copy(src_hbm.at[pl.ds(base, _RPW)], rows_v)

    @pl.loop(0, 4)
    def _(blk):
        pltpu.sync_copy(
            idx_hbm.at[pl.ds(wid * _IDXROWS_PW + blk * 8, 8)], idx_v)
        for j in range(8):
            r0 = pl.multiple_of(blk * 1024 + j * 128, 128)
            pltpu.sync_copy(rows_v.at[pl.ds(r0, 128)], out_hbm.at[idx_v.at[j]])


@functools.cache
def _sc_permute_kernel():
    return pl.kernel(
        _sc_permute_body,
        mesh=plsc.VectorSubcoreMesh(core_axis_name="c", subcore_axis_name="s"),
        out_type=jax.ShapeDtypeStruct((_TOT, _PW), jnp.float32),
        scratch_types=[
            pltpu.VMEM((_RPW, _PW), jnp.float32),
            pltpu.VMEM((8, 128), jnp.int32),
        ],
        compiler_params=pltpu.CompilerParams(use_tc_tiling_on_sc=False),
    )


def _permute_rows(src, idx):
    return _sc_permute_kernel()(src, idx)


def _main_body(xs_ref, cen_ref, cnt_ref, bnd_ref, w1_ref, w2_ref, w3_ref,
               w4a_ref, w4b_ref, enc_ref, acc_ref):
    f32 = jnp.float32
    b = pl.program_id(0)
    acc_ref[...] = jnp.full((_NSEG, 128 + 768), _NEG, f32)
    for c in range(_NCHUNK):
        xc4 = xs_ref[0, pl.ds(c * _CH, _CH), :]
        sw = xc4[:, 0:3]
        pidi = xc4[:, 3:4].astype(jnp.int32)
        iota = jax.lax.broadcasted_iota(jnp.int32, (_CH, _NSEG), 1)
        onehot = (iota == pidi).astype(f32)
        cen_pts = jnp.dot(onehot, cen_ref[0], preferred_element_type=f32)
        aug = jnp.concatenate([sw, sw - cen_pts], axis=1)
        h = jax.nn.relu(jnp.dot(aug, w1_ref[...], preferred_element_type=f32))
        h = jax.nn.relu(jnp.dot(h, w2_ref[...], preferred_element_type=f32))
        h = jax.nn.relu(jnp.dot(h, w3_ref[...], preferred_element_type=f32))
        g = jnp.dot(h, w4a_ref[...], preferred_element_type=f32)
        hg = jnp.concatenate([h, g], axis=1)

        lo = bnd_ref[b, 0, c]
        hi = bnd_ref[b, 1, c]

        def seg_step(s, _):
            m = pidi == s
            cand = jnp.max(jnp.where(m, hg, _NEG), axis=0, keepdims=True)
            acc_ref[pl.ds(s, 1), :] = jnp.maximum(acc_ref[pl.ds(s, 1), :], cand)
            return 0

        jax.lax.fori_loop(lo, hi + 1, seg_step, 0)

    overall = jnp.maximum(acc_ref[:, :128], 0.0)
    m_part = acc_ref[:, 128:]
    cvec = jnp.dot(overall, w4b_ref[...], preferred_element_type=f32)
    enc = jax.nn.relu(m_part + cvec)
    occ = cnt_ref[0] > 0.0
    enc_ref[0] = jnp.where(occ, enc, 0.0)


def _bn_body(enc_ref, w_ref, b_ref, out_ref):
    f32 = jnp.float32
    bsz = enc_ref.shape[0]
    nelem = float(bsz * enc_ref.shape[2])
    s1 = jnp.zeros((_NSEG, 1), f32)
    s2 = jnp.zeros((_NSEG, 1), f32)
    for b in range(bsz):
        e = enc_ref[b]
        s1 = s1 + jnp.sum(e, axis=1, keepdims=True)
        s2 = s2 + jnp.sum(e * e, axis=1, keepdims=True)
    mu = s1 / nelem
    var = s2 / nelem - mu * mu
    scale = w_ref[...] * jax.lax.rsqrt(var + _EPS)
    shift = b_ref[...] - mu * scale
    for b in range(bsz):
        out_ref[b] = enc_ref[b] * scale + shift


@jax.jit
def kernel(x, W1, W2, W3, W4, bn_weight, bn_bias):
    w4a = W4[:128]
    w4b = W4[128:]

    packed, cen, cnt, seg, bnd = pl.pallas_call(
        _prep1_body,
        grid=(_B, _NPCH),
        in_specs=[pl.BlockSpec((1, _PCH, 3), lambda b, c: (b, c, 0))],
        out_specs=[
            pl.BlockSpec((1, _PCH, _PW), lambda b, c: (b, c, 0)),
            pl.BlockSpec((1, _NSEG, 3), lambda b, c: (b, 0, 0)),
            pl.BlockSpec((1, _NSEG, 1), lambda b, c: (b, 0, 0)),
            pl.BlockSpec((1, _NSEG, 1), lambda b, c: (b, 0, 0)),
            pl.BlockSpec((1, 2, _NCHUNK), lambda b, c: (b, 0, 0)),
        ],
        out_shape=[
            jax.ShapeDtypeStruct((_B, _N, _PW), jnp.float32),
            jax.ShapeDtypeStruct((_B, _NSEG, 3), jnp.float32),
            jax.ShapeDtypeStruct((_B, _NSEG, 1), jnp.float32),
            jax.ShapeDtypeStruct((_B, _NSEG, 1), jnp.float32),
            jax.ShapeDtypeStruct((_B, 2, _NCHUNK), jnp.int32),
        ],
        scratch_shapes=[
            pltpu.VMEM((_NSEG, 1), jnp.float32),
            pltpu.VMEM((_NSEG, 3), jnp.float32),
        ],
    )(x)

    dest = pl.pallas_call(
        _prep2_body,
        grid=(_B, _NPCH),
        in_specs=[
            pl.BlockSpec((1, _PCH, 3), lambda b, c: (b, c, 0)),
            pl.BlockSpec((1, _NSEG, 1), lambda b, c: (b, 0, 0)),
        ],
        out_specs=pl.BlockSpec((1, _PCH, 1), lambda b, c: (b, c, 0)),
        out_shape=jax.ShapeDtypeStruct((_B, _N, 1), jnp.int32),
        scratch_shapes=[pltpu.VMEM((_NSEG, 1), jnp.float32)],
    )(x, seg)

    sorted_rows = _permute_rows(
        packed.reshape(_TOT, _PW), dest.reshape(_TOT // 128, 128))

    enc = pl.pallas_call(
        _main_body,
        grid=(_B,),
        in_specs=[
            pl.BlockSpec((1, _N, _PW), lambda b: (b, 0, 0)),
            pl.BlockSpec((1, _NSEG, 3), lambda b: (b, 0, 0)),
            pl.BlockSpec((1, _NSEG, 1), lambda b: (b, 0, 0)),
            pl.BlockSpec(memory_space=pltpu.SMEM),
            pl.BlockSpec((6, 32), lambda b: (0, 0)),
            pl.BlockSpec((32, 64), lambda b: (0, 0)),
            pl.BlockSpec((64, 128), lambda b: (0, 0)),
            pl.BlockSpec((128, 768), lambda b: (0, 0)),
            pl.BlockSpec((128, 768), lambda b: (0, 0)),
        ],
        out_specs=pl.BlockSpec((1, _NSEG, 768), lambda b: (b, 0, 0)),
        out_shape=jax.ShapeDtypeStruct((_B, _NSEG, 768), jnp.float32),
        scratch_shapes=[pltpu.VMEM((_NSEG, 128 + 768), jnp.float32)],
    )(sorted_rows.reshape(_B, _N, _PW), cen, cnt, bnd, W1, W2, W3, w4a, w4b)

    out = pl.pallas_call(
        _bn_body,
        out_shape=jax.ShapeDtypeStruct((_B, _NSEG, 768), jnp.float32),
    )(enc, bn_weight.reshape(_NSEG, 1), bn_bias.reshape(_NSEG, 1))
    return out


# speedup vs baseline: 3.7699x; 1.1733x over previous
import functools

import jax
import jax.numpy as jnp
from jax.experimental import pallas as pl
from jax.experimental.pallas import tpu as pltpu
from jax.experimental.pallas import tpu_sc as plsc

_GRID = 10
_NSEG = _GRID * _GRID
_INTERVAL = 0.2
_EPS = 1e-5
_B = 8
_N = 16384
_TOT = _B * _N
_NEG = -1e30

_PCH = 512
_NPCH = _N // _PCH
_SB = 128
_NSB = _PCH // _SB
_CH = 2048
_NCHUNK = _N // _CH

_NW = 32
_RPW = _TOT // _NW
_IDXROWS_PW = _RPW // 128
_PW = 16


def _bins(xc):
    f32 = jnp.float32
    y = xc[:, 0:1]
    z = xc[:, 1:2]
    xx = xc[:, 2:3]
    swapped = jnp.concatenate([z, y, xx], axis=1)
    ybin = jnp.floor(jnp.clip(y + 1.0, 0.0, 1.99) / _INTERVAL)
    xbin = jnp.floor(jnp.clip(xx + 1.0, 0.0, 1.99) / _INTERVAL)
    pidf = ybin * float(_GRID) + xbin
    pidi = pidf.astype(jnp.int32)
    n = xc.shape[0]
    iota = jax.lax.broadcasted_iota(jnp.int32, (n, _NSEG), 1)
    onehot = (iota == pidi).astype(f32)
    return swapped, pidf, pidi, onehot


_HI = jax.lax.Precision.HIGHEST


def _prep1_body(x_ref, packed_ref, cen_ref, cnt_ref, seg_ref, bnd_ref,
                base_ref, sums_ref):
    f32 = jnp.float32
    c = pl.program_id(1)

    swapped, pidf, _, onehot = _bins(x_ref[0])
    packed_ref[0] = jnp.concatenate(
        [swapped, pidf, jnp.zeros((_PCH, _PW - 4), f32)], axis=1)
    chunk_cnt_row = jnp.sum(onehot, axis=0, keepdims=True)
    chunk_sums = jax.lax.dot_general(
        onehot, swapped, (((0,), (0,)), ((), ())), preferred_element_type=f32)

    @pl.when(c == 0)
    def _():
        base_ref[...] = jnp.zeros((1, _NSEG), f32)
        sums_ref[...] = jnp.zeros((_NSEG, 3), f32)

    base_ref[...] += chunk_cnt_row
    sums_ref[...] += chunk_sums

    @pl.when(c == _NPCH - 1)
    def _():
        ri100 = jax.lax.broadcasted_iota(jnp.int32, (_NSEG, _NSEG), 0)
        ci100 = jax.lax.broadcasted_iota(jnp.int32, (_NSEG, _NSEG), 1)
        lt100 = (ci100 < ri100).astype(f32)
        eye100 = (ri100 == ci100).astype(f32)
        base = jnp.round(jax.lax.dot_general(
            eye100, base_ref[...], (((1,), (1,)), ((), ())),
            precision=_HI, preferred_element_type=f32))
        cnt_ref[0] = base
        cen_ref[0] = sums_ref[...] / jnp.maximum(base, 1.0)
        seg_start = jnp.round(jnp.dot(lt100, base, precision=_HI,
                                      preferred_element_type=f32))
        seg_ref[0] = jax.lax.dot_general(
            seg_start, eye100, (((0,), (0,)), ((), ())),
            precision=_HI, preferred_element_type=f32)
        ends_i = (seg_start + base).astype(jnp.int32)
        pvec = jax.lax.broadcasted_iota(jnp.int32, (1, _NCHUNK), 1) * _CH
        los = jnp.sum((ends_i <= pvec).astype(jnp.int32), axis=0,
                      keepdims=True)
        his = jnp.sum((ends_i <= pvec + (_CH - 1)).astype(jnp.int32), axis=0,
                      keepdims=True)
        bnd_ref[0, 0:1, :] = los
        bnd_ref[0, 1:2, :] = his


def _prep2_body(p_ref, seg_ref, dest_ref, base_ref):
    f32 = jnp.float32
    b = pl.program_id(0)
    c = pl.program_id(1)
    pidi = p_ref[0][:, 3:4].astype(jnp.int32)
    iota = jax.lax.broadcasted_iota(jnp.int32, (_PCH, _NSEG), 1)
    onehot = (iota == pidi).astype(f32)

    @pl.when(c == 0)
    def _():
        base_ref[...] = jnp.zeros((1, _NSEG), f32)

    ri = jax.lax.broadcasted_iota(jnp.int32, (_SB, _SB), 0)
    ci = jax.lax.broadcasted_iota(jnp.int32, (_SB, _SB), 1)
    lt = (ci < ri).astype(f32)
    run = base_ref[...] + seg_ref[0]
    parts = []
    for k in range(_NSB):
        oh = onehot[k * _SB:(k + 1) * _SB]
        prefix = jnp.dot(lt, oh, precision=_HI, preferred_element_type=f32)
        parts.append(jnp.round(
            jnp.sum((prefix + run) * oh, axis=1, keepdims=True)))
        run = run + jnp.sum(oh, axis=0, keepdims=True)
    dest = jnp.concatenate(parts, axis=0)
    dest_ref[0] = dest.astype(jnp.int32) + b * _N
    base_ref[...] = run - seg_ref[0]


def _sc_permute_body(src_hbm, idx_hbm, out_hbm, rows_v, idx_v):
    wid = jax.lax.axis_index("s") * 2 + jax.lax.axis_index("c")
    base = wid * _RPW
    pltpu.sync_copy(src_hbm.at[pl.ds(base, _RPW)], rows_v)

    @pl.loop(0, 4)
    def _(blk):
        pltpu.sync_copy(
            idx_hbm.at[pl.ds(wid * _IDXROWS_PW + blk * 8, 8)], idx_v)
        for j in range(8):
            r0 = pl.multiple_of(blk * 1024 + j * 128, 128)
            pltpu.sync_copy(rows_v.at[pl.ds(r0, 128)], out_hbm.at[idx_v.at[j]])


@functools.cache
def _sc_permute_kernel():
    return pl.kernel(
        _sc_permute_body,
        mesh=plsc.VectorSubcoreMesh(core_axis_name="c", subcore_axis_name="s"),
        out_type=jax.ShapeDtypeStruct((_TOT, _PW), jnp.float32),
        scratch_types=[
            pltpu.VMEM((_RPW, _PW), jnp.float32),
            pltpu.VMEM((8, 128), jnp.int32),
        ],
        compiler_params=pltpu.CompilerParams(use_tc_tiling_on_sc=False),
    )


def _permute_rows(src, idx):
    return _sc_permute_kernel()(src, idx)


def _main_body(xs_ref, cen_ref, cnt_ref, bnd_ref, w1_ref, w2_ref, w3_ref,
               w4a_ref, w4b_ref, enc_ref, acc_ref):
    f32 = jnp.float32
    b = pl.program_id(0)
    acc_ref[...] = jnp.full((_NSEG, 128 + 768), _NEG, f32)
    for c in range(_NCHUNK):
        xc4 = xs_ref[0, pl.ds(c * _CH, _CH), :]
        sw = xc4[:, 0:3]
        pidi = xc4[:, 3:4].astype(jnp.int32)
        iota = jax.lax.broadcasted_iota(jnp.int32, (_CH, _NSEG), 1)
        onehot = (iota == pidi).astype(f32)
        cen_pts = jnp.dot(onehot, cen_ref[0], preferred_element_type=f32)
        aug = jnp.concatenate([sw, sw - cen_pts], axis=1)
        h = jax.nn.relu(jnp.dot(aug, w1_ref[...], preferred_element_type=f32))
        h = jax.nn.relu(jnp.dot(h, w2_ref[...], preferred_element_type=f32))
        h = jax.nn.relu(jnp.dot(h, w3_ref[...], preferred_element_type=f32))
        g = jnp.dot(h, w4a_ref[...], preferred_element_type=f32)
        hg = jnp.concatenate([h, g], axis=1)

        lo = bnd_ref[b, 0, c]
        hi = bnd_ref[b, 1, c]

        def seg_step(s, _):
            m = pidi == s
            cand = jnp.max(jnp.where(m, hg, _NEG), axis=0, keepdims=True)
            acc_ref[pl.ds(s, 1), :] = jnp.maximum(acc_ref[pl.ds(s, 1), :], cand)
            return 0

        jax.lax.fori_loop(lo, hi + 1, seg_step, 0)

    overall = jnp.maximum(acc_ref[:, :128], 0.0)
    m_part = acc_ref[:, 128:]
    cvec = jnp.dot(overall, w4b_ref[...], preferred_element_type=f32)
    enc = jax.nn.relu(m_part + cvec)
    occ = cnt_ref[0] > 0.0
    enc_ref[0] = jnp.where(occ, enc, 0.0)


def _bn_body(enc_ref, w_ref, b_ref, out_ref):
    f32 = jnp.float32
    bsz = enc_ref.shape[0]
    nelem = float(bsz * enc_ref.shape[2])
    s1 = jnp.zeros((_NSEG, 1), f32)
    s2 = jnp.zeros((_NSEG, 1), f32)
    for b in range(bsz):
        e = enc_ref[b]
        s1 = s1 + jnp.sum(e, axis=1, keepdims=True)
        s2 = s2 + jnp.sum(e * e, axis=1, keepdims=True)
    mu = s1 / nelem
    var = s2 / nelem - mu * mu
    scale = w_ref[...] * jax.lax.rsqrt(var + _EPS)
    shift = b_ref[...] - mu * scale
    for b in range(bsz):
        out_ref[b] = enc_ref[b] * scale + shift


@jax.jit
def kernel(x, W1, W2, W3, W4, bn_weight, bn_bias):
    w4a = W4[:128]
    w4b = W4[128:]

    packed, cen, cnt, seg, bnd = pl.pallas_call(
        _prep1_body,
        grid=(_B, _NPCH),
        in_specs=[pl.BlockSpec((1, _PCH, 3), lambda b, c: (b, c, 0))],
        out_specs=[
            pl.BlockSpec((1, _PCH, _PW), lambda b, c: (b, c, 0)),
            pl.BlockSpec((1, _NSEG, 3), lambda b, c: (b, 0, 0)),
            pl.BlockSpec((1, _NSEG, 1), lambda b, c: (b, 0, 0)),
            pl.BlockSpec((1, 1, _NSEG), lambda b, c: (b, 0, 0)),
            pl.BlockSpec((1, 2, _NCHUNK), lambda b, c: (b, 0, 0)),
        ],
        out_shape=[
            jax.ShapeDtypeStruct((_B, _N, _PW), jnp.float32),
            jax.ShapeDtypeStruct((_B, _NSEG, 3), jnp.float32),
            jax.ShapeDtypeStruct((_B, _NSEG, 1), jnp.float32),
            jax.ShapeDtypeStruct((_B, 1, _NSEG), jnp.float32),
            jax.ShapeDtypeStruct((_B, 2, _NCHUNK), jnp.int32),
        ],
        scratch_shapes=[
            pltpu.VMEM((1, _NSEG), jnp.float32),
            pltpu.VMEM((_NSEG, 3), jnp.float32),
        ],
    )(x)

    dest = pl.pallas_call(
        _prep2_body,
        grid=(_B, _NPCH),
        in_specs=[
            pl.BlockSpec((1, _PCH, _PW), lambda b, c: (b, c, 0)),
            pl.BlockSpec((1, 1, _NSEG), lambda b, c: (b, 0, 0)),
        ],
        out_specs=pl.BlockSpec((1, _PCH, 1), lambda b, c: (b, c, 0)),
        out_shape=jax.ShapeDtypeStruct((_B, _N, 1), jnp.int32),
        scratch_shapes=[pltpu.VMEM((1, _NSEG), jnp.float32)],
    )(packed, seg)

    sorted_rows = _permute_rows(
        packed.reshape(_TOT, _PW), dest.reshape(_TOT // 128, 128))

    enc = pl.pallas_call(
        _main_body,
        grid=(_B,),
        in_specs=[
            pl.BlockSpec((1, _N, _PW), lambda b: (b, 0, 0)),
            pl.BlockSpec((1, _NSEG, 3), lambda b: (b, 0, 0)),
            pl.BlockSpec((1, _NSEG, 1), lambda b: (b, 0, 0)),
            pl.BlockSpec(memory_space=pltpu.SMEM),
            pl.BlockSpec((6, 32), lambda b: (0, 0)),
            pl.BlockSpec((32, 64), lambda b: (0, 0)),
            pl.BlockSpec((64, 128), lambda b: (0, 0)),
            pl.BlockSpec((128, 768), lambda b: (0, 0)),
            pl.BlockSpec((128, 768), lambda b: (0, 0)),
        ],
        out_specs=pl.BlockSpec((1, _NSEG, 768), lambda b: (b, 0, 0)),
        out_shape=jax.ShapeDtypeStruct((_B, _NSEG, 768), jnp.float32),
        scratch_shapes=[pltpu.VMEM((_NSEG, 128 + 768), jnp.float32)],
    )(sorted_rows.reshape(_B, _N, _PW), cen, cnt, bnd, W1, W2, W3, w4a, w4b)

    out = pl.pallas_call(
        _bn_body,
        out_shape=jax.ShapeDtypeStruct((_B, _NSEG, 768), jnp.float32),
    )(enc, bn_weight.reshape(_NSEG, 1), bn_bias.reshape(_NSEG, 1))
    return out


# speedup vs baseline: 5.7625x; 1.5286x over previous
import functools

import jax
import jax.numpy as jnp
from jax.experimental import pallas as pl
from jax.experimental.pallas import tpu as pltpu
from jax.experimental.pallas import tpu_sc as plsc

_GRID = 10
_NSEG = _GRID * _GRID
_INTERVAL = 0.2
_EPS = 1e-5
_B = 8
_N = 16384
_TOT = _B * _N
_NEG = -1e30

_PCH = 1024
_NPCH = _N // _PCH
_SB = 128
_NSB = _PCH // _SB
_CH = 128
_NCHUNK = _N // _CH
_MCH = 2048
_NM = _N // _MCH
_WPM = _MCH // _CH

_NW = 32
_RPW = _TOT // _NW
_IDXROWS_PW = _RPW // 128
_PW = 16


def _bins(xc):
    f32 = jnp.float32
    y = xc[:, 0:1]
    z = xc[:, 1:2]
    xx = xc[:, 2:3]
    swapped = jnp.concatenate([z, y, xx], axis=1)
    ybin = jnp.floor(jnp.clip(y + 1.0, 0.0, 1.99) / _INTERVAL)
    xbin = jnp.floor(jnp.clip(xx + 1.0, 0.0, 1.99) / _INTERVAL)
    pidf = ybin * float(_GRID) + xbin
    pidi = pidf.astype(jnp.int32)
    n = xc.shape[0]
    iota = jax.lax.broadcasted_iota(jnp.int32, (n, _NSEG), 1)
    onehot = (iota == pidi).astype(f32)
    return swapped, pidf, pidi, onehot


_HI = jax.lax.Precision.HIGHEST


def _prep1_body(x_ref, packed_ref, cen_ref, cnt_ref, seg_ref, bnd_ref,
                base_ref, sums_ref):
    f32 = jnp.float32
    c = pl.program_id(1)

    swapped, pidf, _, onehot = _bins(x_ref[0])
    packed_ref[...] = jnp.concatenate(
        [swapped, pidf, jnp.zeros((_PCH, _PW - 4), f32)], axis=1)
    chunk_cnt_row = jnp.sum(onehot, axis=0, keepdims=True)
    chunk_sums = jax.lax.dot_general(
        onehot, swapped, (((0,), (0,)), ((), ())), preferred_element_type=f32)

    @pl.when(c == 0)
    def _():
        base_ref[...] = jnp.zeros((1, _NSEG), f32)
        sums_ref[...] = jnp.zeros((_NSEG, 3), f32)

    base_ref[...] += chunk_cnt_row
    sums_ref[...] += chunk_sums

    @pl.when(c == _NPCH - 1)
    def _():
        ri100 = jax.lax.broadcasted_iota(jnp.int32, (_NSEG, _NSEG), 0)
        ci100 = jax.lax.broadcasted_iota(jnp.int32, (_NSEG, _NSEG), 1)
        lt100 = (ci100 < ri100).astype(f32)
        eye100 = (ri100 == ci100).astype(f32)
        base = jnp.round(jax.lax.dot_general(
            eye100, base_ref[...], (((1,), (1,)), ((), ())),
            precision=_HI, preferred_element_type=f32))
        cnt_ref[0] = base
        cen_ref[0] = sums_ref[...] / jnp.maximum(base, 1.0)
        seg_start = jnp.round(jnp.dot(lt100, base, precision=_HI,
                                      preferred_element_type=f32))
        seg_ref[0] = jax.lax.dot_general(
            seg_start, eye100, (((0,), (0,)), ((), ())),
            precision=_HI, preferred_element_type=f32)
        ends_i = (seg_start + base).astype(jnp.int32)
        pvec = jax.lax.broadcasted_iota(jnp.int32, (1, _NCHUNK), 1) * _CH
        los = jnp.sum((ends_i <= pvec).astype(jnp.int32), axis=0,
                      keepdims=True)
        his = jnp.sum((ends_i <= pvec + (_CH - 1)).astype(jnp.int32), axis=0,
                      keepdims=True)
        bnd_ref[0, 0:1, :] = los
        bnd_ref[0, 1:2, :] = his


def _prep2_body(p_ref, seg_ref, dest_ref, base_ref):
    f32 = jnp.float32
    b = pl.program_id(0)
    c = pl.program_id(1)
    pidi = p_ref[:, 3:4].astype(jnp.int32)
    iota = jax.lax.broadcasted_iota(jnp.int32, (_PCH, _NSEG), 1)
    onehot = (iota == pidi).astype(f32)

    @pl.when(c == 0)
    def _():
        base_ref[...] = jnp.zeros((1, _NSEG), f32)

    ri = jax.lax.broadcasted_iota(jnp.int32, (_SB, _SB), 0)
    ci = jax.lax.broadcasted_iota(jnp.int32, (_SB, _SB), 1)
    lt = (ci < ri).astype(f32)
    eye = (ri == ci).astype(f32)
    run = base_ref[...] + seg_ref[0]
    parts = []
    for k in range(_NSB):
        oh = onehot[k * _SB:(k + 1) * _SB]
        prefix = jnp.dot(lt, oh, precision=_HI, preferred_element_type=f32)
        d = jnp.round(jnp.sum((prefix + run) * oh, axis=1, keepdims=True))
        parts.append(jax.lax.dot_general(
            d, eye, (((0,), (0,)), ((), ())),
            precision=_HI, preferred_element_type=f32))
        run = run + jnp.sum(oh, axis=0, keepdims=True)
    dest = jnp.concatenate(parts, axis=0)
    dest_ref[...] = dest.astype(jnp.int32) + b * _N
    base_ref[...] = run - seg_ref[0]


def _sc_permute_body(src_hbm, idx_hbm, out_hbm, rows_v, idx_v):
    wid = jax.lax.axis_index("s") * 2 + jax.lax.axis_index("c")
    base = wid * _RPW
    pltpu.sync_copy(src_hbm.at[pl.ds(base, _RPW)], rows_v)

    @pl.loop(0, 4)
    def _(blk):
        pltpu.sync_copy(
            idx_hbm.at[pl.ds(wid * _IDXROWS_PW + blk * 8, 8)], idx_v)
        for j in range(8):
            r0 = pl.multiple_of(blk * 1024 + j * 128, 128)
            pltpu.sync_copy(rows_v.at[pl.ds(r0, 128)], out_hbm.at[idx_v.at[j]])


@functools.cache
def _sc_permute_kernel():
    return pl.kernel(
        _sc_permute_body,
        mesh=plsc.VectorSubcoreMesh(core_axis_name="c", subcore_axis_name="s"),
        out_type=jax.ShapeDtypeStruct((_TOT, _PW), jnp.float32),
        scratch_types=[
            pltpu.VMEM((_RPW, _PW), jnp.float32),
            pltpu.VMEM((8, 128), jnp.int32),
        ],
        compiler_params=pltpu.CompilerParams(use_tc_tiling_on_sc=False),
    )


def _permute_rows(src, idx):
    return _sc_permute_kernel()(src, idx)


def _main_body(xs_ref, cen_ref, cnt_ref, bnd_ref, w1_ref, w2_ref, w3_ref,
               w4a_ref, w4b_ref, enc_ref, acc_ref, hg_ref):
    f32 = jnp.float32
    b = pl.program_id(0)
    acc_ref[...] = jnp.full((_NSEG, 128 + 768), _NEG, f32)

    for mc in range(_NM):
        xc4 = xs_ref[pl.ds(mc * _MCH, _MCH), :]
        sw = xc4[:, 0:3]
        pidi = xc4[:, 3:4].astype(jnp.int32)
        iota = jax.lax.broadcasted_iota(jnp.int32, (_MCH, _NSEG), 1)
        onehot = (iota == pidi).astype(f32)
        cen_pts = jnp.dot(onehot, cen_ref[0], preferred_element_type=f32)
        aug = jnp.concatenate([sw, sw - cen_pts], axis=1)
        h = jax.nn.relu(jnp.dot(aug, w1_ref[...], preferred_element_type=f32))
        h = jax.nn.relu(jnp.dot(h, w2_ref[...], preferred_element_type=f32))
        h = jax.nn.relu(jnp.dot(h, w3_ref[...], preferred_element_type=f32))
        g = jnp.dot(h, w4a_ref[...], preferred_element_type=f32)
        hg_ref[...] = jnp.concatenate([h, g], axis=1)

        def win_step(w, _):
            pw = xs_ref[pl.ds(mc * _MCH + w * _CH, _CH), 3:4].astype(jnp.int32)
            lo = bnd_ref[b, 0, mc * _WPM + w]
            hi = bnd_ref[b, 1, mc * _WPM + w]

            def seg_step(s, _):
                m = pw == s
                hgw = hg_ref[pl.ds(w * _CH, _CH), :]
                cand = jnp.max(jnp.where(m, hgw, _NEG), axis=0, keepdims=True)
                acc_ref[pl.ds(s, 1), :] = jnp.maximum(
                    acc_ref[pl.ds(s, 1), :], cand)
                return 0

            jax.lax.fori_loop(lo, hi + 1, seg_step, 0)
            return 0

        jax.lax.fori_loop(0, _WPM, win_step, 0)

    overall = jnp.maximum(acc_ref[:, :128], 0.0)
    m_part = acc_ref[:, 128:]
    cvec = jnp.dot(overall, w4b_ref[...], preferred_element_type=f32)
    enc = jax.nn.relu(m_part + cvec)
    occ = cnt_ref[0] > 0.0
    enc_ref[0] = jnp.where(occ, enc, 0.0)


def _bn_body(enc_ref, w_ref, b_ref, out_ref):
    f32 = jnp.float32
    bsz = enc_ref.shape[0]
    nelem = float(bsz * enc_ref.shape[2])
    s1 = jnp.zeros((_NSEG, 1), f32)
    s2 = jnp.zeros((_NSEG, 1), f32)
    for b in range(bsz):
        e = enc_ref[b]
        s1 = s1 + jnp.sum(e, axis=1, keepdims=True)
        s2 = s2 + jnp.sum(e * e, axis=1, keepdims=True)
    mu = s1 / nelem
    var = s2 / nelem - mu * mu
    scale = w_ref[...] * jax.lax.rsqrt(var + _EPS)
    shift = b_ref[...] - mu * scale
    for b in range(bsz):
        out_ref[b] = enc_ref[b] * scale + shift


@jax.jit
def kernel(x, W1, W2, W3, W4, bn_weight, bn_bias):
    w4a = W4[:128]
    w4b = W4[128:]

    packed, cen, cnt, seg, bnd = pl.pallas_call(
        _prep1_body,
        grid=(_B, _NPCH),
        in_specs=[pl.BlockSpec((1, _PCH, 3), lambda b, c: (b, c, 0))],
        out_specs=[
            pl.BlockSpec((_PCH, _PW), lambda b, c: (b * _NPCH + c, 0)),
            pl.BlockSpec((1, _NSEG, 3), lambda b, c: (b, 0, 0)),
            pl.BlockSpec((1, _NSEG, 1), lambda b, c: (b, 0, 0)),
            pl.BlockSpec((1, 1, _NSEG), lambda b, c: (b, 0, 0)),
            pl.BlockSpec((1, 2, _NCHUNK), lambda b, c: (b, 0, 0)),
        ],
        out_shape=[
            jax.ShapeDtypeStruct((_TOT, _PW), jnp.float32),
            jax.ShapeDtypeStruct((_B, _NSEG, 3), jnp.float32),
            jax.ShapeDtypeStruct((_B, _NSEG, 1), jnp.float32),
            jax.ShapeDtypeStruct((_B, 1, _NSEG), jnp.float32),
            jax.ShapeDtypeStruct((_B, 2, _NCHUNK), jnp.int32),
        ],
        scratch_shapes=[
            pltpu.VMEM((1, _NSEG), jnp.float32),
            pltpu.VMEM((_NSEG, 3), jnp.float32),
        ],
    )(x)

    dest = pl.pallas_call(
        _prep2_body,
        grid=(_B, _NPCH),
        in_specs=[
            pl.BlockSpec((_PCH, _PW), lambda b, c: (b * _NPCH + c, 0)),
            pl.BlockSpec((1, 1, _NSEG), lambda b, c: (b, 0, 0)),
        ],
        out_specs=pl.BlockSpec((_NSB, 128), lambda b, c: (b * _NPCH + c, 0)),
        out_shape=jax.ShapeDtypeStruct((_TOT // 128, 128), jnp.int32),
        scratch_shapes=[pltpu.VMEM((1, _NSEG), jnp.float32)],
    )(packed, seg)

    sorted_rows = _permute_rows(packed, dest)

    enc = pl.pallas_call(
        _main_body,
        grid=(_B,),
        in_specs=[
            pl.BlockSpec((_N, _PW), lambda b: (b, 0)),
            pl.BlockSpec((1, _NSEG, 3), lambda b: (b, 0, 0)),
            pl.BlockSpec((1, _NSEG, 1), lambda b: (b, 0, 0)),
            pl.BlockSpec(memory_space=pltpu.SMEM),
            pl.BlockSpec((6, 32), lambda b: (0, 0)),
            pl.BlockSpec((32, 64), lambda b: (0, 0)),
            pl.BlockSpec((64, 128), lambda b: (0, 0)),
            pl.BlockSpec((128, 768), lambda b: (0, 0)),
            pl.BlockSpec((128, 768), lambda b: (0, 0)),
        ],
        out_specs=pl.BlockSpec((1, _NSEG, 768), lambda b: (b, 0, 0)),
        out_shape=jax.ShapeDtypeStruct((_B, _NSEG, 768), jnp.float32),
        scratch_shapes=[
            pltpu.VMEM((_NSEG, 128 + 768), jnp.float32),
            pltpu.VMEM((_MCH, 128 + 768), jnp.float32),
        ],
    )(sorted_rows, cen, cnt, bnd, W1, W2, W3, w4a, w4b)

    out = pl.pallas_call(
        _bn_body,
        out_shape=jax.ShapeDtypeStruct((_B, _NSEG, 768), jnp.float32),
    )(enc, bn_weight.reshape(_NSEG, 1), bn_bias.reshape(_NSEG, 1))
    return out


# speedup vs baseline: 6.0879x; 1.0565x over previous
import functools

import jax
import jax.numpy as jnp
from jax.experimental import pallas as pl
from jax.experimental.pallas import tpu as pltpu
from jax.experimental.pallas import tpu_sc as plsc

_GRID = 10
_NSEG = _GRID * _GRID
_INTERVAL = 0.2
_EPS = 1e-5
_B = 8
_N = 16384
_TOT = _B * _N
_NEG = -1e30

_PCH = 1024
_NPCH = _N // _PCH
_SB = 128
_NSB = _PCH // _SB
_CH = 128
_NCHUNK = _N // _CH
_MCH = 2048
_NM = _N // _MCH
_WPM = _MCH // _CH

_NW = 32
_RPW = _TOT // _NW
_IDXROWS_PW = _RPW // 128
_PW = 16


def _bins(xc):
    f32 = jnp.float32
    y = xc[:, 0:1]
    z = xc[:, 1:2]
    xx = xc[:, 2:3]
    swapped = jnp.concatenate([z, y, xx], axis=1)
    ybin = jnp.floor(jnp.clip(y + 1.0, 0.0, 1.99) / _INTERVAL)
    xbin = jnp.floor(jnp.clip(xx + 1.0, 0.0, 1.99) / _INTERVAL)
    pidf = ybin * float(_GRID) + xbin
    pidi = pidf.astype(jnp.int32)
    n = xc.shape[0]
    iota = jax.lax.broadcasted_iota(jnp.int32, (n, _NSEG), 1)
    onehot = (iota == pidi).astype(f32)
    return swapped, pidf, pidi, onehot


_HI = jax.lax.Precision.HIGHEST


def _prep1_body(x_ref, packed_ref, cen_ref, cnt_ref, seg_ref, bnd_ref,
                base_ref, sums_ref):
    f32 = jnp.float32
    c = pl.program_id(1)

    swapped, pidf, _, onehot = _bins(x_ref[0])
    packed_ref[...] = jnp.concatenate(
        [swapped, pidf, jnp.zeros((_PCH, _PW - 4), f32)], axis=1)
    chunk_cnt_row = jnp.sum(onehot, axis=0, keepdims=True)
    chunk_sums = jax.lax.dot_general(
        onehot, swapped, (((0,), (0,)), ((), ())), preferred_element_type=f32)

    @pl.when(c == 0)
    def _():
        base_ref[...] = jnp.zeros((1, _NSEG), f32)
        sums_ref[...] = jnp.zeros((_NSEG, 3), f32)

    base_ref[...] += chunk_cnt_row
    sums_ref[...] += chunk_sums

    @pl.when(c == _NPCH - 1)
    def _():
        ri100 = jax.lax.broadcasted_iota(jnp.int32, (_NSEG, _NSEG), 0)
        ci100 = jax.lax.broadcasted_iota(jnp.int32, (_NSEG, _NSEG), 1)
        lt100 = (ci100 < ri100).astype(f32)
        eye100 = (ri100 == ci100).astype(f32)
        base = jnp.round(jax.lax.dot_general(
            eye100, base_ref[...], (((1,), (1,)), ((), ())),
            precision=_HI, preferred_element_type=f32))
        cnt_ref[0] = base
        cen_ref[0] = sums_ref[...] / jnp.maximum(base, 1.0)
        seg_start = jnp.round(jnp.dot(lt100, base, precision=_HI,
                                      preferred_element_type=f32))
        seg_ref[0] = jax.lax.dot_general(
            seg_start, eye100, (((0,), (0,)), ((), ())),
            precision=_HI, preferred_element_type=f32)
        ends_i = (seg_start + base).astype(jnp.int32)
        pvec = jax.lax.broadcasted_iota(jnp.int32, (1, _NCHUNK), 1) * _CH
        los = jnp.sum((ends_i <= pvec).astype(jnp.int32), axis=0,
                      keepdims=True)
        his = jnp.sum((ends_i <= pvec + (_CH - 1)).astype(jnp.int32), axis=0,
                      keepdims=True)
        bnd_ref[0, 0:1, :] = los
        bnd_ref[0, 1:2, :] = his


def _prep2_body(p_ref, seg_ref, dest_ref, base_ref):
    f32 = jnp.float32
    b = pl.program_id(0)
    c = pl.program_id(1)
    pidi = p_ref[:, 3:4].astype(jnp.int32)
    iota = jax.lax.broadcasted_iota(jnp.int32, (_PCH, _NSEG), 1)
    onehot = (iota == pidi).astype(f32)

    @pl.when(c == 0)
    def _():
        base_ref[...] = jnp.zeros((1, _NSEG), f32)

    ri = jax.lax.broadcasted_iota(jnp.int32, (_SB, _SB), 0)
    ci = jax.lax.broadcasted_iota(jnp.int32, (_SB, _SB), 1)
    lt = (ci < ri).astype(f32)
    eye = (ri == ci).astype(f32)
    run = base_ref[...] + seg_ref[0]
    parts = []
    for k in range(_NSB):
        oh = onehot[k * _SB:(k + 1) * _SB]
        prefix = jnp.dot(lt, oh, precision=_HI, preferred_element_type=f32)
        parts.append(jnp.round(
            jnp.sum((prefix + run) * oh, axis=1, keepdims=True)))
        run = run + jnp.sum(oh, axis=0, keepdims=True)
    dmat = jnp.concatenate(parts, axis=1)
    dest = jax.lax.dot_general(
        dmat, eye, (((0,), (0,)), ((), ())),
        precision=_HI, preferred_element_type=f32)
    dest_ref[...] = dest.astype(jnp.int32) + b * _N
    base_ref[...] = run - seg_ref[0]


def _sc_permute_body(src_hbm, idx_hbm, out_hbm, rows_v, idx_v):
    wid = jax.lax.axis_index("s") * 2 + jax.lax.axis_index("c")
    base = wid * _RPW
    pltpu.sync_copy(src_hbm.at[pl.ds(base, _RPW)], rows_v)

    @pl.loop(0, 4)
    def _(blk):
        pltpu.sync_copy(
            idx_hbm.at[pl.ds(wid * _IDXROWS_PW + blk * 8, 8)], idx_v)
        for j in range(8):
            r0 = pl.multiple_of(blk * 1024 + j * 128, 128)
            pltpu.sync_copy(rows_v.at[pl.ds(r0, 128)], out_hbm.at[idx_v.at[j]])


@functools.cache
def _sc_permute_kernel():
    return pl.kernel(
        _sc_permute_body,
        mesh=plsc.VectorSubcoreMesh(core_axis_name="c", subcore_axis_name="s"),
        out_type=jax.ShapeDtypeStruct((_TOT, _PW), jnp.float32),
        scratch_types=[
            pltpu.VMEM((_RPW, _PW), jnp.float32),
            pltpu.VMEM((8, 128), jnp.int32),
        ],
        compiler_params=pltpu.CompilerParams(use_tc_tiling_on_sc=False),
    )


def _permute_rows(src, idx):
    return _sc_permute_kernel()(src, idx)


def _main_body(xs_ref, cen_ref, cnt_ref, bnd_ref, w1_ref, w2_ref, w3_ref,
               w4a_ref, w4b_ref, enc_ref, acc_ref, hg_ref):
    f32 = jnp.float32
    b = pl.program_id(0)
    acc_ref[...] = jnp.full((_NSEG, 128 + 768), _NEG, f32)

    for mc in range(_NM):
        xc4 = xs_ref[pl.ds(mc * _MCH, _MCH), :]
        sw = xc4[:, 0:3]
        pidi = xc4[:, 3:4].astype(jnp.int32)
        iota = jax.lax.broadcasted_iota(jnp.int32, (_MCH, _NSEG), 1)
        onehot = (iota == pidi).astype(f32)
        cen_pts = jnp.dot(onehot, cen_ref[0], preferred_element_type=f32)
        aug = jnp.concatenate([sw, sw - cen_pts], axis=1)
        h = jax.nn.relu(jnp.dot(aug, w1_ref[...], preferred_element_type=f32))
        h = jax.nn.relu(jnp.dot(h, w2_ref[...], preferred_element_type=f32))
        h = jax.nn.relu(jnp.dot(h, w3_ref[...], preferred_element_type=f32))
        g = jnp.dot(h, w4a_ref[...], preferred_element_type=f32)
        hg_ref[...] = jnp.concatenate([h, g], axis=1)

        def win_step(w, _):
            pw = xs_ref[pl.ds(mc * _MCH + w * _CH, _CH), 3:4].astype(jnp.int32)
            lo = bnd_ref[b, 0, mc * _WPM + w]
            hi = bnd_ref[b, 1, mc * _WPM + w]

            def seg_step(s, _):
                m = pw == s
                hgw = hg_ref[pl.ds(w * _CH, _CH), :]
                cand = jnp.max(jnp.where(m, hgw, _NEG), axis=0, keepdims=True)
                acc_ref[pl.ds(s, 1), :] = jnp.maximum(
                    acc_ref[pl.ds(s, 1), :], cand)
                return 0

            jax.lax.fori_loop(lo, hi + 1, seg_step, 0)
            return 0

        jax.lax.fori_loop(0, _WPM, win_step, 0)

    overall = jnp.maximum(acc_ref[:, :128], 0.0)
    m_part = acc_ref[:, 128:]
    cvec = jnp.dot(overall, w4b_ref[...], preferred_element_type=f32)
    enc = jax.nn.relu(m_part + cvec)
    occ = cnt_ref[0] > 0.0
    enc_ref[0] = jnp.where(occ, enc, 0.0)


def _bn_body(enc_ref, w_ref, b_ref, out_ref):
    f32 = jnp.float32
    bsz = enc_ref.shape[0]
    nelem = float(bsz * enc_ref.shape[2])
    s1 = jnp.zeros((_NSEG, 1), f32)
    s2 = jnp.zeros((_NSEG, 1), f32)
    for b in range(bsz):
        e = enc_ref[b]
        s1 = s1 + jnp.sum(e, axis=1, keepdims=True)
        s2 = s2 + jnp.sum(e * e, axis=1, keepdims=True)
    mu = s1 / nelem
    var = s2 / nelem - mu * mu
    scale = w_ref[...] * jax.lax.rsqrt(var + _EPS)
    shift = b_ref[...] - mu * scale
    for b in range(bsz):
        out_ref[b] = enc_ref[b] * scale + shift


@jax.jit
def kernel(x, W1, W2, W3, W4, bn_weight, bn_bias):
    w4a = W4[:128]
    w4b = W4[128:]

    packed, cen, cnt, seg, bnd = pl.pallas_call(
        _prep1_body,
        grid=(_B, _NPCH),
        in_specs=[pl.BlockSpec((1, _PCH, 3), lambda b, c: (b, c, 0))],
        out_specs=[
            pl.BlockSpec((_PCH, _PW), lambda b, c: (b * _NPCH + c, 0)),
            pl.BlockSpec((1, _NSEG, 3), lambda b, c: (b, 0, 0)),
            pl.BlockSpec((1, _NSEG, 1), lambda b, c: (b, 0, 0)),
            pl.BlockSpec((1, 1, _NSEG), lambda b, c: (b, 0, 0)),
            pl.BlockSpec((1, 2, _NCHUNK), lambda b, c: (b, 0, 0)),
        ],
        out_shape=[
            jax.ShapeDtypeStruct((_TOT, _PW), jnp.float32),
            jax.ShapeDtypeStruct((_B, _NSEG, 3), jnp.float32),
            jax.ShapeDtypeStruct((_B, _NSEG, 1), jnp.float32),
            jax.ShapeDtypeStruct((_B, 1, _NSEG), jnp.float32),
            jax.ShapeDtypeStruct((_B, 2, _NCHUNK), jnp.int32),
        ],
        scratch_shapes=[
            pltpu.VMEM((1, _NSEG), jnp.float32),
            pltpu.VMEM((_NSEG, 3), jnp.float32),
        ],
    )(x)

    dest = pl.pallas_call(
        _prep2_body,
        grid=(_B, _NPCH),
        in_specs=[
            pl.BlockSpec((_PCH, _PW), lambda b, c: (b * _NPCH + c, 0)),
            pl.BlockSpec((1, 1, _NSEG), lambda b, c: (b, 0, 0)),
        ],
        out_specs=pl.BlockSpec((_NSB, 128), lambda b, c: (b * _NPCH + c, 0)),
        out_shape=jax.ShapeDtypeStruct((_TOT // 128, 128), jnp.int32),
        scratch_shapes=[pltpu.VMEM((1, _NSEG), jnp.float32)],
    )(packed, seg)

    sorted_rows = _permute_rows(packed, dest)

    enc = pl.pallas_call(
        _main_body,
        grid=(_B,),
        in_specs=[
            pl.BlockSpec((_N, _PW), lambda b: (b, 0)),
            pl.BlockSpec((1, _NSEG, 3), lambda b: (b, 0, 0)),
            pl.BlockSpec((1, _NSEG, 1), lambda b: (b, 0, 0)),
            pl.BlockSpec(memory_space=pltpu.SMEM),
            pl.BlockSpec((6, 32), lambda b: (0, 0)),
            pl.BlockSpec((32, 64), lambda b: (0, 0)),
            pl.BlockSpec((64, 128), lambda b: (0, 0)),
            pl.BlockSpec((128, 768), lambda b: (0, 0)),
            pl.BlockSpec((128, 768), lambda b: (0, 0)),
        ],
        out_specs=pl.BlockSpec((1, _NSEG, 768), lambda b: (b, 0, 0)),
        out_shape=jax.ShapeDtypeStruct((_B, _NSEG, 768), jnp.float32),
        scratch_shapes=[
            pltpu.VMEM((_NSEG, 128 + 768), jnp.float32),
            pltpu.VMEM((_MCH, 128 + 768), jnp.float32),
        ],
    )(sorted_rows, cen, cnt, bnd, W1, W2, W3, w4a, w4b)

    out = pl.pallas_call(
        _bn_body,
        out_shape=jax.ShapeDtypeStruct((_B, _NSEG, 768), jnp.float32),
    )(enc, bn_weight.reshape(_NSEG, 1), bn_bias.reshape(_NSEG, 1))
    return out
